# baseline (speedup 1.0000x reference)
import functools

import jax
import jax.numpy as jnp
from jax import lax
from jax.experimental import pallas as pl
from jax.experimental.pallas import tpu as pltpu
from jax.experimental.pallas import tpu_sc as plsc

N_NODES = 10000
N_EDGES = 320000
CH = 128

NC = 2
NS = 16
NW = NC * NS
CHUNK = 128
NCHUNKS = 80
HALF = NCHUNKS // 2
E_PAD = NW * NCHUNKS * CHUNK
N_PAD = 10240
ROWS_PER_TILE = N_PAD // NS
ZROWS = 128
LANES = 16


_mesh = plsc.VectorSubcoreMesh(core_axis_name="c", subcore_axis_name="s")


@functools.partial(
    pl.kernel,
    out_type=jax.ShapeDtypeStruct((NC, N_PAD, CH), jnp.float32),
    mesh=_mesh,
    scratch_types=[
        pltpu.VMEM((HALF, CHUNK), jnp.int32),
        pltpu.VMEM((HALF, CHUNK), jnp.int32),
        pltpu.VMEM((2, CHUNK, CH), jnp.float32),
        pltpu.VMEM_SHARED((N_PAD, CH), jnp.float32),
        pltpu.SemaphoreType.DMA,
        pltpu.SemaphoreType.DMA,
    ],
)
def _sc_segment_sum(row_hbm, col_hbm, x_hbm, out_hbm,
                    rowv, colv, rows, acc, isem, gsem):
    c = lax.axis_index("c")
    s = lax.axis_index("s")
    wid = c * NS + s

    def _zero_row(i, carry):
        zero = jnp.zeros((LANES,), jnp.float32)
        for j in range(CH // LANES):
            rows[0, i, pl.ds(j * LANES, LANES)] = zero
        return carry
    lax.fori_loop(0, CHUNK, _zero_row, 0)
    for k in range(ROWS_PER_TILE // CHUNK):
        pltpu.sync_copy(rows.at[0],
                        acc.at[pl.ds(s * ROWS_PER_TILE + k * CHUNK, CHUNK)])
    plsc.subcore_barrier()

    for h in range(2):
        base = wid * NCHUNKS + h * HALF
        pltpu.async_copy(row_hbm.at[pl.ds(base, HALF)], rowv, isem)
        pltpu.async_copy(col_hbm.at[pl.ds(base, HALF)], colv, isem)
        pltpu.make_async_copy(row_hbm.at[pl.ds(base, HALF)], rowv, isem).wait()
        pltpu.make_async_copy(col_hbm.at[pl.ds(base, HALF)], colv, isem).wait()

        pltpu.async_copy(x_hbm.at[colv.at[0]], rows.at[0], gsem)

        def _chunk(i, carry):
            buf = lax.rem(i, 2)
            nbuf = lax.rem(i + 1, 2)
            nxt = lax.rem(i + 1, HALF)
            pltpu.make_async_copy(x_hbm.at[colv.at[i]], rows.at[buf],
                                  gsem).wait()
            pltpu.async_copy(x_hbm.at[colv.at[nxt]], rows.at[nbuf], gsem)
            pltpu.sync_copy(rows.at[buf], acc.at[rowv.at[i]], add=True)
            return carry
        lax.fori_loop(0, HALF, _chunk, 0)
        pltpu.make_async_copy(x_hbm.at[colv.at[0]], rows.at[lax.rem(HALF, 2)],
                              gsem).wait()

    plsc.subcore_barrier()
    pltpu.sync_copy(acc.at[pl.ds(s * ROWS_PER_TILE, ROWS_PER_TILE)],
                    out_hbm.at[c, pl.ds(s * ROWS_PER_TILE, ROWS_PER_TILE)])


def _tc_matmul_body(p_ref, w_ref, o_ref):
    seg = p_ref[0, :N_NODES, :] + p_ref[1, :N_NODES, :]
    o_ref[...] = lax.dot_general(
        seg, w_ref[...], (((1,), (1,)), ((), ())),
        preferred_element_type=jnp.float32,
        precision=lax.Precision.HIGHEST)


def kernel(x, edge_index, edge_attr, W_fc, W_edge, W_att):
    npad = E_PAD - N_EDGES
    row = jnp.concatenate(
        [edge_index[0].astype(jnp.int32),
         jnp.full((npad,), N_NODES, jnp.int32)]).reshape(NW * NCHUNKS, CHUNK)
    col = jnp.concatenate(
        [edge_index[1].astype(jnp.int32),
         jnp.zeros((npad,), jnp.int32)]).reshape(NW * NCHUNKS, CHUNK)
    partials = _sc_segment_sum(row, col, x)
    z = pl.pallas_call(
        _tc_matmul_body,
        out_shape=jax.ShapeDtypeStruct((N_NODES, CH), jnp.float32),
    )(partials, W_fc)
    return z
